# baseline (speedup 1.0000x reference)
import jax
import jax.numpy as jnp
from jax.experimental import pallas as pl
from jax.experimental.pallas import tpu as pltpu

_H = 64


def _fused_gru_kernel(scal_ref, wt_ref, bt_ref, x_ref, h_ref, d_ref, w_ref,
                      b_ref, out_ref, hnew_ref):
    i = pl.program_id(0)
    nsteps = pl.num_programs(0)

    @pl.when(i == 0)
    def _init():
        out_ref[...] = jnp.full(out_ref.shape, -jnp.inf, jnp.float32)

    x = x_ref[...]
    h = h_ref[...]
    xh = jnp.concatenate([x, h], axis=1)
    g = jax.lax.dot_general(
        xh, w_ref[...], (((1,), (0,)), ((), ())),
        preferred_element_type=jnp.float32) + b_ref[...]
    r = jax.nn.sigmoid(g[:, 0:_H])
    z = jax.nn.sigmoid(g[:, _H:2 * _H])
    n = jnp.tanh(g[:, 2 * _H:3 * _H] + r * g[:, 3 * _H:4 * _H])
    h_all = (1.0 - z) * n + z * h

    dmax = jnp.max(d_ref[...], axis=1, keepdims=True)
    mask = (dmax > 0.0) & (scal_ref[1] > 0.0)
    hnew_ref[...] = jnp.where(mask, h_all, 0.0)

    block_max = jnp.max(jnp.where(mask, h_all, -jnp.inf), axis=0,
                        keepdims=True)
    out_ref[...] = jnp.maximum(out_ref[...], block_max)

    @pl.when(i == nsteps - 1)
    def _finalize():
        inv = 1.0 / jnp.log(scal_ref[0] + jnp.exp(1.0))
        out_ref[...] = out_ref[...] + jnp.tanh(inv * wt_ref[...] + bt_ref[...])


def kernel(interval, t, co_embeddings, divided, no_embeddings,
           unrelated_embeddings, is_last, hidden_state, W_ih, W_hh, b_ih,
           b_hh, W_t, b_t):
    N, G = co_embeddings.shape
    H = W_hh.shape[1]
    if hidden_state is None:
        hidden_state = jnp.zeros((N, H), co_embeddings.dtype)

    WiT = W_ih.T
    WhT = W_hh.T
    Wbig = jnp.concatenate([
        jnp.concatenate([WiT[:, :H], WiT[:, H:2 * H], WiT[:, 2 * H:],
                         jnp.zeros((G, H), jnp.float32)], axis=1),
        jnp.concatenate([WhT[:, :H], WhT[:, H:2 * H],
                         jnp.zeros((H, H), jnp.float32), WhT[:, 2 * H:]],
                        axis=1),
    ], axis=0)
    bbig = jnp.concatenate([
        b_ih[:H] + b_hh[:H], b_ih[H:2 * H] + b_hh[H:2 * H],
        b_ih[2 * H:], b_hh[2 * H:]])[None, :]

    scalars = jnp.stack([
        jnp.asarray(interval, jnp.float32),
        jnp.logical_not(is_last).astype(jnp.float32)])
    wt_row = W_t.T.astype(jnp.float32)
    bt_row = b_t[None, :]

    B = 2000
    grid = N // B

    out_small, h_new = pl.pallas_call(
        _fused_gru_kernel,
        grid=(grid,),
        in_specs=[
            pl.BlockSpec(memory_space=pltpu.SMEM),
            pl.BlockSpec((1, H), lambda i: (0, 0)),
            pl.BlockSpec((1, H), lambda i: (0, 0)),
            pl.BlockSpec((B, G), lambda i: (i, 0)),
            pl.BlockSpec((B, H), lambda i: (i, 0)),
            pl.BlockSpec((B, 3), lambda i: (i, 0)),
            pl.BlockSpec((G + H, 4 * H), lambda i: (0, 0)),
            pl.BlockSpec((1, 4 * H), lambda i: (0, 0)),
        ],
        out_specs=[
            pl.BlockSpec((1, H), lambda i: (0, 0)),
            pl.BlockSpec((B, H), lambda i: (i, 0)),
        ],
        out_shape=[
            jax.ShapeDtypeStruct((1, H), jnp.float32),
            jax.ShapeDtypeStruct((N, H), jnp.float32),
        ],
        compiler_params=pltpu.CompilerParams(
            dimension_semantics=("arbitrary",)),
    )(scalars, wt_row, bt_row, co_embeddings, hidden_state, divided, Wbig,
      bbig)

    return (out_small.reshape(H), h_new)
